# baseline (speedup 1.0000x reference)
import functools

import jax
import jax.numpy as jnp
from jax import lax
from jax.experimental import pallas as pl
from jax.experimental.pallas import tpu as pltpu
from jax.experimental.pallas import tpu_sc as plsc

NC = 2
NS = 16
NW = NC * NS
L = 16


def _rsqrt16(x):
    i = lax.bitcast_convert_type(x, jnp.int32)
    i = jnp.int32(0x5F3759DF) - (i >> 1)
    y = lax.bitcast_convert_type(i, jnp.float32)
    xh = x * 0.5
    for _ in range(3):
        y = y * (1.5 - xh * y * y)
    return y


def _make_sc_h01_kernel(B, S, V, D):
    assert B % NW == 0 and D % L == 0 and V % L == 0
    rows_per_w = B // NW
    assert rows_per_w >= 4 and rows_per_w % 2 == 0
    VP = D + 8
    n_groups = -(-S // L)
    full_groups = S // L
    nid = rows_per_w * S
    stage_n = -(-(nid + L) // L) * L

    mesh = plsc.VectorSubcoreMesh(core_axis_name="c", subcore_axis_name="s",
                                  num_cores=NC, num_subcores=NS)
    out_t = jax.ShapeDtypeStruct((B, S, D), jnp.float32)

    @functools.partial(
        pl.kernel,
        out_type=(out_t, out_t),
        mesh=mesh,
        compiler_params=pltpu.CompilerParams(needs_layout_passes=False,
                                             use_tc_tiling_on_sc=False),
        scratch_types=[
            pltpu.VMEM((V, D), jnp.float32),
            pltpu.VMEM((D,), jnp.float32),
            pltpu.VMEM((D,), jnp.float32),
            pltpu.VMEM((V * VP,), jnp.float32),
            pltpu.VMEM((V * VP,), jnp.float32),
            pltpu.VMEM((L,), jnp.float32),
            pltpu.VMEM((stage_n,), jnp.int32),
            pltpu.VMEM((S, D), jnp.float32),
            pltpu.VMEM((S, D), jnp.float32),
            pltpu.VMEM((S, D), jnp.float32),
            pltpu.VMEM((S, D), jnp.float32),
            pltpu.SemaphoreType.DMA,
            pltpu.SemaphoreType.DMA,
        ],
    )
    def sc_kernel(ids_hbm, tab_hbm, g_hbm, bt_hbm, h0_hbm, h1_hbm,
                  tab_v, g_v, bt_v, t0p, t1p, red_v, stage_v,
                  h0b0, h1b0, h0b1, h1b1, sem_o0, sem_o1):
        wid = lax.axis_index("s") * NC + lax.axis_index("c")
        h0b = [h0b0, h0b1]
        h1b = [h1b0, h1b1]
        sem_o = [sem_o0, sem_o1]
        pltpu.sync_copy(tab_hbm, tab_v)
        pltpu.sync_copy(g_hbm, g_v)
        pltpu.sync_copy(bt_hbm, bt_v)
        pltpu.sync_copy(ids_hbm.at[pl.ds(wid * nid, nid)],
                        stage_v.at[pl.ds(0, nid)])

        lane = lax.iota(jnp.int32, L)
        zero16 = jnp.zeros((L,), jnp.float32)
        g_vecs = [g_v[pl.ds(j * L, L)] for j in range(D // L)]
        bt_vecs = [bt_v[pl.ds(j * L, L)] for j in range(D // L)]
        g_s = [g_vecs[d // L][d % L] for d in range(D)]
        bt_s = [bt_vecs[d // L][d % L] for d in range(D)]

        def splat_i(v):
            return jnp.broadcast_to(v, (L,)).astype(jnp.int32)

        for g2 in range(V // L):
            idv = g2 * L + lane
            s0 = zero16
            s1 = zero16
            for d in range(D):
                x = plsc.load_gather(tab_v, [idv, splat_i(d)])
                plsc.store_scatter(t0p, [idv * VP + d], x)
                s0 = s0 + x
                s1 = s1 + x * x
            mu = s0 * (1.0 / D)
            var = s1 * (1.0 / D) - mu * mu
            r0 = _rsqrt16(var + 1e-5)
            for d in range(D):
                x = plsc.load_gather(tab_v, [idv, splat_i(d)])
                h1 = (x - mu) * r0 * g_s[d] + bt_s[d]
                plsc.store_scatter(t1p, [idv * VP + d], h1)

        def compute_row(r, p):
            tot = zero16
            for g in range(n_groups):
                v = stage_v[pl.ds(r * S + g * L, L)]
                nv = min(L, S - g * L)
                if nv < L:
                    v = jnp.where(lane < nv, v, 0)
                tot = tot + v.astype(jnp.float32)
            for kk in (8, 4, 2, 1):
                red_v[...] = tot
                tot = tot + plsc.load_gather(red_v, [lane ^ kk])
            bias = tot * (0.05 / S)

            def emit_token(tok, id_s):
                base = id_s * VP
                for d0 in range(0, D, L):
                    sl = pl.ds(d0, L)
                    h0b[p][tok, sl] = t0p[pl.ds(base + d0, L)] + bias
                    h1b[p][tok, sl] = t1p[pl.ds(base + d0, L)]

            def grp(g, _):
                ids_vec = stage_v[pl.ds(r * S + g * L, L)]
                for i in range(L):
                    emit_token(g * L + i, ids_vec[i])
                return _

            lax.fori_loop(0, full_groups, grp, None)
            if full_groups < n_groups:
                gt = full_groups
                ids_vec = stage_v[pl.ds(r * S + gt * L, L)]
                for i in range(S - gt * L):
                    emit_token(gt * L + i, ids_vec[i])

        def fire_out(r, p):
            b = wid * rows_per_w + r
            pltpu.async_copy(h0b[p], h0_hbm.at[b], sem_o[p])
            pltpu.async_copy(h1b[p], h1_hbm.at[b], sem_o[p])

        def wait_out(r, p):
            b = wid * rows_per_w + r
            pltpu.make_async_copy(h0b[p], h0_hbm.at[b], sem_o[p]).wait()
            pltpu.make_async_copy(h1b[p], h1_hbm.at[b], sem_o[p]).wait()

        compute_row(0, 0)
        fire_out(0, 0)
        compute_row(1, 1)
        fire_out(1, 1)

        def step(r, p):
            wait_out(r - 2, p)
            compute_row(r, p)
            fire_out(r, p)

        def row_pair(i, _):
            step(2 * i + 2, 0)
            step(2 * i + 3, 1)
            return _

        lax.fori_loop(0, (rows_per_w - 2) // 2, row_pair, None)
        wait_out(rows_per_w - 2, 0)
        wait_out(rows_per_w - 1, 1)

    return sc_kernel


def _make_tc_h12_kernel(N, V, D, BK=2048):
    assert N % BK == 0
    steps = N // BK

    def body(ids_ref, tab_ref, g_ref, bt_ref, h2_ref, t2_s):
        @pl.when(pl.program_id(0) == 0)
        def _():
            t = tab_ref[...]
            g = g_ref[...]
            bt = bt_ref[...]
            mu = jnp.mean(t, axis=1, keepdims=True)
            var = jnp.mean((t - mu) ** 2, axis=1, keepdims=True)
            t1 = (t - mu) * lax.rsqrt(var + 1e-5) * g + bt
            mu1 = jnp.mean(t1, axis=1, keepdims=True)
            var1 = jnp.mean((t1 - mu1) ** 2, axis=1, keepdims=True)
            t2_s[...] = (t1 - mu1) * lax.rsqrt(var1 + 1e-5) * g + bt

        ids = ids_ref[...]
        oh = (ids == lax.broadcasted_iota(jnp.int32, (BK, V), 1))
        oh = oh.astype(jnp.float32)
        h2_ref[...] = jnp.dot(oh, t2_s[...],
                              preferred_element_type=jnp.float32)

    out_t = jax.ShapeDtypeStruct((N, D), jnp.float32)
    return pl.pallas_call(
        body,
        grid=(steps,),
        in_specs=[
            pl.BlockSpec((BK, 1), lambda i: (i, 0)),
            pl.BlockSpec((V, D), lambda i: (0, 0)),
            pl.BlockSpec((1, D), lambda i: (0, 0)),
            pl.BlockSpec((1, D), lambda i: (0, 0)),
        ],
        out_specs=pl.BlockSpec((BK, D), lambda i: (i, 0)),
        out_shape=out_t,
        scratch_shapes=[pltpu.VMEM((V, D), jnp.float32)],
    )


def kernel(input_ids, table, gamma, beta):
    B, S = input_ids.shape
    V, D = table.shape
    ids_flat = input_ids.reshape(-1).astype(jnp.int32)
    tab32 = table.astype(jnp.float32)
    g32 = gamma.astype(jnp.float32)
    bt32 = beta.astype(jnp.float32)
    sc = _make_sc_h01_kernel(B, S, V, D)
    h0, h1 = sc(ids_flat, tab32, g32, bt32)
    tc = _make_tc_h12_kernel(B * S, V, D)
    h2 = tc(ids_flat.reshape(-1, 1), tab32,
            g32.reshape(1, D), bt32.reshape(1, D))
    return h0, h1, h2.reshape((B, S, D))


# speedup vs baseline: 1.4603x; 1.4603x over previous
import functools

import jax
import jax.numpy as jnp
from jax import lax
from jax.experimental import pallas as pl
from jax.experimental.pallas import tpu as pltpu
from jax.experimental.pallas import tpu_sc as plsc

NC = 2
NS = 16
NW = NC * NS
L = 16


def _make_sc_h0_kernel(B, S, V, D):
    assert B % NW == 0 and D % L == 0 and V % L == 0
    rows_per_w = B // NW
    assert rows_per_w >= 4 and rows_per_w % 2 == 0
    VP = D + 8
    n_groups = -(-S // L)
    full_groups = S // L
    nid = rows_per_w * S
    stage_n = -(-(nid + L) // L) * L

    mesh = plsc.VectorSubcoreMesh(core_axis_name="c", subcore_axis_name="s",
                                  num_cores=NC, num_subcores=NS)
    out_t = jax.ShapeDtypeStruct((B, S, D), jnp.float32)

    @functools.partial(
        pl.kernel,
        out_type=out_t,
        mesh=mesh,
        compiler_params=pltpu.CompilerParams(needs_layout_passes=False,
                                             use_tc_tiling_on_sc=False),
        scratch_types=[
            pltpu.VMEM((V, D), jnp.float32),
            pltpu.VMEM((V * VP,), jnp.float32),
            pltpu.VMEM((L,), jnp.float32),
            pltpu.VMEM((stage_n,), jnp.int32),
            pltpu.VMEM((S, D), jnp.float32),
            pltpu.VMEM((S, D), jnp.float32),
            pltpu.SemaphoreType.DMA,
            pltpu.SemaphoreType.DMA,
        ],
    )
    def sc_kernel(ids_hbm, tab_hbm, h0_hbm,
                  tab_v, t0p, red_v, stage_v, h0b0, h0b1, sem_o0, sem_o1):
        wid = lax.axis_index("s") * NC + lax.axis_index("c")
        h0b = [h0b0, h0b1]
        sem_o = [sem_o0, sem_o1]
        pltpu.sync_copy(tab_hbm, tab_v)
        pltpu.sync_copy(ids_hbm.at[pl.ds(wid * nid, nid)],
                        stage_v.at[pl.ds(0, nid)])

        lane = lax.iota(jnp.int32, L)
        zero16 = jnp.zeros((L,), jnp.float32)

        def splat_i(v):
            return jnp.broadcast_to(v, (L,)).astype(jnp.int32)

        for g2 in range(V // L):
            idv = g2 * L + lane
            for d in range(D):
                x = plsc.load_gather(tab_v, [idv, splat_i(d)])
                plsc.store_scatter(t0p, [idv * VP + d], x)

        def compute_row(r, p):
            tot = zero16
            for g in range(n_groups):
                v = stage_v[pl.ds(r * S + g * L, L)]
                nv = min(L, S - g * L)
                if nv < L:
                    v = jnp.where(lane < nv, v, 0)
                tot = tot + v.astype(jnp.float32)
            for kk in (8, 4, 2, 1):
                red_v[...] = tot
                tot = tot + plsc.load_gather(red_v, [lane ^ kk])
            bias = tot * (0.05 / S)

            def emit_token(tok, id_s):
                base = id_s * VP
                for d0 in range(0, D, L):
                    h0b[p][tok, pl.ds(d0, L)] = t0p[pl.ds(base + d0, L)] + bias

            def grp(g, _):
                ids_vec = stage_v[pl.ds(r * S + g * L, L)]
                for i in range(L):
                    emit_token(g * L + i, ids_vec[i])
                return _

            lax.fori_loop(0, full_groups, grp, None)
            if full_groups < n_groups:
                gt = full_groups
                ids_vec = stage_v[pl.ds(r * S + gt * L, L)]
                for i in range(S - gt * L):
                    emit_token(gt * L + i, ids_vec[i])

        def fire_out(r, p):
            b = wid * rows_per_w + r
            pltpu.async_copy(h0b[p], h0_hbm.at[b], sem_o[p])

        def wait_out(r, p):
            b = wid * rows_per_w + r
            pltpu.make_async_copy(h0b[p], h0_hbm.at[b], sem_o[p]).wait()

        compute_row(0, 0)
        fire_out(0, 0)
        compute_row(1, 1)
        fire_out(1, 1)

        def step(r, p):
            wait_out(r - 2, p)
            compute_row(r, p)
            fire_out(r, p)

        def row_pair(i, _):
            step(2 * i + 2, 0)
            step(2 * i + 3, 1)
            return _

        lax.fori_loop(0, (rows_per_w - 2) // 2, row_pair, None)
        wait_out(rows_per_w - 2, 0)
        wait_out(rows_per_w - 1, 1)

    return sc_kernel


def _make_tc_h12_kernel(N, V, D, BK=8192):
    assert N % BK == 0
    steps = N // BK

    def body(ids_ref, tab_ref, g_ref, bt_ref, h1_ref, h2_ref, t12_s):
        @pl.when(pl.program_id(0) == 0)
        def _():
            t = tab_ref[...]
            g = g_ref[...]
            bt = bt_ref[...]
            mu = jnp.mean(t, axis=1, keepdims=True)
            var = jnp.mean((t - mu) ** 2, axis=1, keepdims=True)
            t1 = (t - mu) * lax.rsqrt(var + 1e-5) * g + bt
            mu1 = jnp.mean(t1, axis=1, keepdims=True)
            var1 = jnp.mean((t1 - mu1) ** 2, axis=1, keepdims=True)
            t2 = (t1 - mu1) * lax.rsqrt(var1 + 1e-5) * g + bt
            t12_s[...] = jnp.concatenate([t1, t2], axis=1)

        ids = ids_ref[...]
        oh = (ids == lax.broadcasted_iota(jnp.int32, (BK, V), 1))
        oh = oh.astype(jnp.float32)
        res = jnp.dot(oh, t12_s[...], preferred_element_type=jnp.float32)
        h1_ref[...] = res[:, :D]
        h2_ref[...] = res[:, D:]

    out_t = jax.ShapeDtypeStruct((N, D), jnp.float32)
    return pl.pallas_call(
        body,
        grid=(steps,),
        in_specs=[
            pl.BlockSpec((BK, 1), lambda i: (i, 0)),
            pl.BlockSpec((V, D), lambda i: (0, 0)),
            pl.BlockSpec((1, D), lambda i: (0, 0)),
            pl.BlockSpec((1, D), lambda i: (0, 0)),
        ],
        out_specs=[
            pl.BlockSpec((BK, D), lambda i: (i, 0)),
            pl.BlockSpec((BK, D), lambda i: (i, 0)),
        ],
        out_shape=(out_t, out_t),
        scratch_shapes=[pltpu.VMEM((V, 2 * D), jnp.float32)],
    )


def kernel(input_ids, table, gamma, beta):
    B, S = input_ids.shape
    V, D = table.shape
    ids_flat = input_ids.reshape(-1).astype(jnp.int32)
    tab32 = table.astype(jnp.float32)
    sc = _make_sc_h0_kernel(B, S, V, D)
    h0 = sc(ids_flat, tab32)
    tc = _make_tc_h12_kernel(B * S, V, D)
    h1, h2 = tc(ids_flat.reshape(-1, 1), tab32,
                gamma.astype(jnp.float32).reshape(1, D),
                beta.astype(jnp.float32).reshape(1, D))
    shp = (B, S, D)
    return h0, h1.reshape(shp), h2.reshape(shp)
